# baseline (speedup 1.0000x reference)
import jax
import jax.numpy as jnp
from jax.experimental import pallas as pl
from jax.experimental.pallas import tpu as pltpu

_BT = 4096


def _body(x_ref, o_ref):
    o_ref[...] = x_ref[:, :8]


def kernel(hidden_states, W_router):
    b, s, h = hidden_states.shape
    n_tok = b * s
    x = hidden_states.reshape(n_tok, h)
    grid = (n_tok // _BT,)
    out = pl.pallas_call(
        _body,
        grid=grid,
        in_specs=[pl.BlockSpec((_BT, h), lambda i: (i, 0))],
        out_specs=pl.BlockSpec((_BT, 8), lambda i: (i, 0)),
        out_shape=jax.ShapeDtypeStruct((n_tok, 8), jnp.float32),
    )(x)
    w = out[:, :2].reshape(b, s, 2)
    e = out[:, :2].astype(jnp.int32).reshape(b, s, 2)
    return (w, e, out.reshape(b, s, 8))


# speedup vs baseline: 1.0123x; 1.0123x over previous
import jax
import jax.numpy as jnp
from jax.experimental import pallas as pl
from jax.experimental.pallas import tpu as pltpu

_BT = 512
_NBUF = 4


def _router_body(x_hbm, w_router_ref, logits_ref, w_ref, e_ref, bufs, sems):
    n_tok = x_hbm.shape[0]
    steps = n_tok // _BT
    w_router = w_router_ref[...]
    nexp = w_router.shape[0]

    def start(t, b):
        pltpu.make_async_copy(
            x_hbm.at[pl.ds(t * _BT, _BT)], bufs.at[b], sems.at[b]
        ).start()

    for b in range(_NBUF):
        start(b, b)

    def outer(o, carry):
        for b in range(_NBUF):
            t = o * _NBUF + b
            pltpu.make_async_copy(
                x_hbm.at[pl.ds(t * _BT, _BT)], bufs.at[b], sems.at[b]
            ).wait()
            logits_t = jax.lax.dot_general(
                w_router, bufs[b],
                dimension_numbers=(((1,), (1,)), ((), ())),
                preferred_element_type=jnp.float32,
            )
            logits_ref[pl.ds(t * _BT, _BT)] = logits_t.T
            idx = jax.lax.broadcasted_iota(jnp.int32, logits_t.shape, 0)
            m1 = jnp.max(logits_t, axis=0, keepdims=True)
            a1 = jnp.min(jnp.where(logits_t == m1, idx, nexp), axis=0,
                         keepdims=True)
            masked = jnp.where(idx == a1, -jnp.inf, logits_t)
            m2 = jnp.max(masked, axis=0, keepdims=True)
            a2 = jnp.min(jnp.where(masked == m2, idx, nexp), axis=0,
                         keepdims=True)
            w1 = 1.0 / (1.0 + jnp.exp(m2 - m1))
            w2 = 1.0 - w1
            w_ref[pl.ds(t * _BT, _BT)] = jnp.concatenate([w1, w2], axis=0).T
            e_ref[pl.ds(t * _BT, _BT)] = jnp.concatenate([a1, a2], axis=0).T

            @pl.when(t + _NBUF < steps)
            def _():
                start(t + _NBUF, b)
        return carry

    jax.lax.fori_loop(0, steps // _NBUF, outer, 0)


def kernel(hidden_states, W_router):
    b, s, h = hidden_states.shape
    n_exp = W_router.shape[0]
    n_tok = b * s
    x = hidden_states.reshape(n_tok, h)

    logits, weights, experts = pl.pallas_call(
        _router_body,
        in_specs=[
            pl.BlockSpec(memory_space=pl.ANY),
            pl.BlockSpec(memory_space=pltpu.MemorySpace.VMEM),
        ],
        out_specs=[
            pl.BlockSpec(memory_space=pltpu.MemorySpace.VMEM),
            pl.BlockSpec(memory_space=pltpu.MemorySpace.VMEM),
            pl.BlockSpec(memory_space=pltpu.MemorySpace.VMEM),
        ],
        out_shape=[
            jax.ShapeDtypeStruct((n_tok, n_exp), jnp.float32),
            jax.ShapeDtypeStruct((n_tok, 2), jnp.float32),
            jax.ShapeDtypeStruct((n_tok, 2), jnp.int32),
        ],
        scratch_shapes=[
            pltpu.VMEM((_NBUF, _BT, h), jnp.float32),
            pltpu.SemaphoreType.DMA((_NBUF,)),
        ],
    )(x, W_router)

    return (
        weights.reshape(b, s, 2),
        experts.reshape(b, s, 2),
        logits.reshape(b, s, n_exp),
    )
